# Initial kernel scaffold; baseline (speedup 1.0000x reference)
import jax
import jax.numpy as jnp
from jax.experimental import pallas as pl


def kernel(symbol, positional_encoding):
    raise NotImplementedError("write your pallas kernel here")


# speedup vs baseline: 4.1796x; 4.1796x over previous
import jax
import jax.numpy as jnp
from jax.experimental import pallas as pl

BLOCK_L = 512
D_MODEL = 768
PAD = 0


def _pe_mask_kernel(sym_ref, pe_ref, out_ref):
    pe = pe_ref[...]
    B = out_ref.shape[0]
    for b in range(B):
        mask = (sym_ref[:, pl.ds(b, 1)] != PAD).astype(pe.dtype)
        out_ref[b] = pe * mask


def kernel(symbol, positional_encoding):
    B, L = symbol.shape
    pe = positional_encoding.reshape(positional_encoding.shape[-2:])
    D = pe.shape[-1]
    sym_t = symbol.astype(jnp.int32).T
    num_l = L // BLOCK_L
    return pl.pallas_call(
        _pe_mask_kernel,
        grid=(num_l,),
        in_specs=[
            pl.BlockSpec((BLOCK_L, B), lambda l: (l, 0)),
            pl.BlockSpec((BLOCK_L, D), lambda l: (l, 0)),
        ],
        out_specs=pl.BlockSpec((B, BLOCK_L, D), lambda l: (0, l, 0)),
        out_shape=jax.ShapeDtypeStruct((B, L, D), pe.dtype),
    )(sym_t, pe)


# speedup vs baseline: 4.3368x; 1.0376x over previous
import jax
import jax.numpy as jnp
from jax.experimental import pallas as pl

BLOCK_L = 1024
D_MODEL = 768
PAD = 0


def _pe_mask_kernel(sym_ref, pe_ref, out_ref):
    pe = pe_ref[...]
    B = out_ref.shape[0]
    for b in range(B):
        mask = (sym_ref[:, pl.ds(b, 1)] != PAD).astype(pe.dtype)
        out_ref[b] = pe * mask


def kernel(symbol, positional_encoding):
    B, L = symbol.shape
    pe = positional_encoding.reshape(positional_encoding.shape[-2:])
    D = pe.shape[-1]
    sym_t = symbol.astype(jnp.int32).T
    num_l = L // BLOCK_L
    return pl.pallas_call(
        _pe_mask_kernel,
        grid=(num_l,),
        in_specs=[
            pl.BlockSpec((BLOCK_L, B), lambda l: (l, 0)),
            pl.BlockSpec((BLOCK_L, D), lambda l: (l, 0)),
        ],
        out_specs=pl.BlockSpec((B, BLOCK_L, D), lambda l: (0, l, 0)),
        out_shape=jax.ShapeDtypeStruct((B, L, D), pe.dtype),
    )(sym_t, pe)
